# baseline (speedup 1.0000x reference)
import functools

import jax
import jax.numpy as jnp
from jax import lax
from jax.experimental import pallas as pl
from jax.experimental.pallas import tpu as pltpu
from jax.experimental.pallas import tpu_sc as plsc

_ALPHA = 0.5
_EPS = 0.01
_GAMMA = 0.99
_N_STATES = 100000
_N_ACTIONS = 64
_B = 16384

_NC = 2
_NS = 16
_NW = _NC * _NS
_BPW = _B // _NW
_G = _BPW // 16

_RPW = _N_STATES // _NW
_CR = 125
_NCH = _RPW // _CR
_CAP = 1024
_NLV = _CAP // 16


def _mesh():
    return plsc.VectorSubcoreMesh(
        core_axis_name="c", subcore_axis_name="s",
        num_cores=_NC, num_subcores=_NS)


def _wid():
    return lax.axis_index("s") * _NC + lax.axis_index("c")


def _dyn_gather(v, idx):
    dnums = lax.GatherDimensionNumbers(
        offset_dims=(), collapsed_slice_dims=(0,), start_index_map=(0,))
    return lax.gather(v, idx[:, None], dnums, (1,),
                      mode=lax.GatherScatterMode.PROMISE_IN_BOUNDS)


def _stage1_body(q_hbm, cs_hbm, sn_hbm, rew_hbm, rv_hbm, ra_hbm,
                 act_hbm, flat_hbm, nv_hbm,
                 cs_v, sn_v, rew_v, rv_v, ra_v, rows_cs, rows_sn,
                 act_o, flat_o, nv_o, sem1, sem2):
    base = _wid() * _BPW
    sl_in = pl.ds(base, _BPW)
    pltpu.sync_copy(cs_hbm.at[sl_in], cs_v)
    pltpu.sync_copy(sn_hbm.at[sl_in], sn_v)
    c1 = pltpu.async_copy(q_hbm.at[cs_v], rows_cs, sem1)
    c2 = pltpu.async_copy(q_hbm.at[sn_v], rows_sn, sem2)
    pltpu.sync_copy(rew_hbm.at[sl_in], rew_v)
    pltpu.sync_copy(rv_hbm.at[sl_in], rv_v)
    pltpu.sync_copy(ra_hbm.at[sl_in], ra_v)
    c1.wait()
    c2.wait()

    iot = lax.iota(jnp.int32, 16)
    zeros16 = jnp.zeros((16,), jnp.int32)
    def gbody(g, _):
        sl = pl.ds(g * 16, 16)
        rows16 = g * 16 + iot
        m = plsc.load_gather(rows_cs, [rows16, zeros16])
        m2 = plsc.load_gather(rows_sn, [rows16, zeros16])
        mi = zeros16

        def jbody(j, carry):
            m, mi, m2 = carry
            cj = jnp.full((16,), j, jnp.int32)
            v = plsc.load_gather(rows_cs, [rows16, cj])
            v2 = plsc.load_gather(rows_sn, [rows16, cj])
            better = v > m
            m = jnp.where(better, v, m)
            mi = jnp.where(better, cj, mi)
            m2 = jnp.maximum(m2, v2)
            return m, mi, m2

        m, mi, m2 = lax.fori_loop(1, _N_ACTIONS, jbody, (m, mi, m2))

        acts = jnp.where(rv_v[sl] > _EPS, mi, ra_v[sl])
        q_sa = plsc.load_gather(rows_cs, [rows16, acts])
        nv = q_sa + _ALPHA * (rew_v[sl] + _GAMMA * m2 - q_sa)
        act_o[sl] = acts
        flat_o[sl] = cs_v[sl] * _N_ACTIONS + acts
        nv_o[sl] = nv
        return 0

    lax.fori_loop(0, _G, gbody, 0)

    pltpu.sync_copy(act_o, act_hbm.at[sl_in])
    pltpu.sync_copy(flat_o, flat_hbm.at[sl_in])
    pltpu.sync_copy(nv_o, nv_hbm.at[sl_in])


def _stage2_body(q_hbm, flat_hbm, nv_hbm, out_hbm,
                 flat_v, val_v, loc_list, val_list, buf0, buf1,
                 semi0, semi1, semo0, semo1):
    wid = _wid()
    row0 = wid * _RPW
    lo = row0 * _N_ACTIONS
    hi = lo + _RPW * _N_ACTIONS

    pltpu.sync_copy(flat_hbm, flat_v)
    pltpu.sync_copy(nv_hbm, val_v)

    iot = lax.iota(jnp.int32, 16)
    neg1 = jnp.full((16,), -1, jnp.int32)

    def prebody(i, _):
        loc_list[pl.ds(i * 16, 16)] = neg1
        return 0

    lax.fori_loop(0, _NLV, prebody, 0)

    def fbody(i, cur):
        sl = pl.ds(i * 16, 16)
        fv = flat_v[sl]
        vv = val_v[sl]
        msk = (fv >= lo) & (fv < hi)
        mi32 = msk.astype(jnp.int32)
        pos = jnp.clip(cur + plsc.cumsum(mi32) - 1, 0, _CAP - 1)
        plsc.store_scatter(loc_list, [pos], fv - lo, mask=msk)
        plsc.store_scatter(val_list, [pos], vv, mask=msk)
        return cur + jnp.sum(mi32)

    cnt = lax.fori_loop(0, _B // 16, fbody, jnp.int32(0))
    nvec = (cnt + 15) >> 4

    def kbody(i, _):
        sl = pl.ds(i * 16, 16)
        lv = loc_list[sl]
        dup = lv < -1
        for s in range(1, 16):
            sh = _dyn_gather(lv, jnp.minimum(iot + s, 15))
            dup = dup | ((lv == sh) & (iot < 16 - s))
        loc_list[sl] = jnp.where(dup, neg1, lv)
        return 0

    lax.fori_loop(0, nvec, kbody, 0)

    bufs = (buf0, buf1)
    semis = (semi0, semi1)
    semos = (semo0, semo1)
    cflat = _CR * _N_ACTIONS

    def rows_at(c):
        return pl.ds(lo + c * cflat, cflat)

    in_cp = [None, None]
    out_cp = [None, None]
    in_cp[0] = pltpu.async_copy(q_hbm.at[rows_at(0)], bufs[0], semis[0])
    for c in range(_NCH):
        p = c % 2
        if c + 1 < _NCH:
            nxt = (c + 1) % 2
            if out_cp[nxt] is not None:
                out_cp[nxt].wait()
            in_cp[nxt] = pltpu.async_copy(
                q_hbm.at[rows_at(c + 1)], bufs[nxt], semis[nxt])
        in_cp[p].wait()

        c0 = c * cflat
        buf = bufs[p]

        def pbody(i, _, c0=c0, buf=buf):
            sl = pl.ds(i * 16, 16)
            lv = loc_list[sl]
            vv = val_list[sl]
            msk = (lv >= c0) & (lv < c0 + cflat)
            off = jnp.clip(lv - c0, 0, cflat - 1)
            plsc.store_scatter(buf, [off], vv, mask=msk)
            return 0

        lax.fori_loop(0, nvec, pbody, 0)
        out_cp[p] = pltpu.async_copy(buf, out_hbm.at[rows_at(c)], semos[p])
    out_cp[(_NCH - 1) % 2].wait()
    if _NCH > 1:
        out_cp[_NCH % 2].wait()


def kernel(Q_table, reward, rand_vals, current_state, state_next, rand_actions):
    stage1 = functools.partial(
        pl.kernel,
        out_type=(jax.ShapeDtypeStruct((_B,), jnp.int32),
                  jax.ShapeDtypeStruct((_B,), jnp.int32),
                  jax.ShapeDtypeStruct((_B,), jnp.float32)),
        mesh=_mesh(),
        compiler_params=pltpu.CompilerParams(needs_layout_passes=False, use_tc_tiling_on_sc=False),
        scratch_types=[
            pltpu.VMEM((_BPW,), jnp.int32),
            pltpu.VMEM((_BPW,), jnp.int32),
            pltpu.VMEM((_BPW,), jnp.float32),
            pltpu.VMEM((_BPW,), jnp.float32),
            pltpu.VMEM((_BPW,), jnp.int32),
            pltpu.VMEM((_BPW, _N_ACTIONS), jnp.float32),
            pltpu.VMEM((_BPW, _N_ACTIONS), jnp.float32),
            pltpu.VMEM((_BPW,), jnp.int32),
            pltpu.VMEM((_BPW,), jnp.int32),
            pltpu.VMEM((_BPW,), jnp.float32),
            pltpu.SemaphoreType.DMA,
            pltpu.SemaphoreType.DMA,
        ],
    )(_stage1_body)
    actions, flat_idx, new_vals = stage1(
        Q_table, current_state, state_next, reward, rand_vals, rand_actions)

    stage2 = functools.partial(
        pl.kernel,
        out_type=jax.ShapeDtypeStruct((_N_STATES * _N_ACTIONS,), jnp.float32),
        mesh=_mesh(),
        compiler_params=pltpu.CompilerParams(needs_layout_passes=False, use_tc_tiling_on_sc=False),
        scratch_types=[
            pltpu.VMEM((_B,), jnp.int32),
            pltpu.VMEM((_B,), jnp.float32),
            pltpu.VMEM((_CAP,), jnp.int32),
            pltpu.VMEM((_CAP,), jnp.float32),
            pltpu.VMEM((_CR * _N_ACTIONS,), jnp.float32),
            pltpu.VMEM((_CR * _N_ACTIONS,), jnp.float32),
            pltpu.SemaphoreType.DMA,
            pltpu.SemaphoreType.DMA,
            pltpu.SemaphoreType.DMA,
            pltpu.SemaphoreType.DMA,
        ],
    )(_stage2_body)
    new_Q = stage2(Q_table.reshape(-1), flat_idx, new_vals)
    return actions, new_Q.reshape(_N_STATES, _N_ACTIONS)


# speedup vs baseline: 1.0045x; 1.0045x over previous
import functools

import jax
import jax.numpy as jnp
from jax import lax
from jax.experimental import pallas as pl
from jax.experimental.pallas import tpu as pltpu
from jax.experimental.pallas import tpu_sc as plsc

_ALPHA = 0.5
_EPS = 0.01
_GAMMA = 0.99
_N_STATES = 100000
_N_ACTIONS = 64
_B = 16384

_NC = 2
_NS = 16
_NW = _NC * _NS
_BPW = _B // _NW
_G = _BPW // 16

_RPW = _N_STATES // _NW
_CR = 125
_NCH = _RPW // _CR
_CAP = 1024
_NLV = _CAP // 16


def _mesh():
    return plsc.VectorSubcoreMesh(
        core_axis_name="c", subcore_axis_name="s",
        num_cores=_NC, num_subcores=_NS)


def _wid():
    return lax.axis_index("s") * _NC + lax.axis_index("c")


def _dyn_gather(v, idx):
    dnums = lax.GatherDimensionNumbers(
        offset_dims=(), collapsed_slice_dims=(0,), start_index_map=(0,))
    return lax.gather(v, idx[:, None], dnums, (1,),
                      mode=lax.GatherScatterMode.PROMISE_IN_BOUNDS)


def _stage1_body(q_hbm, cs_hbm, sn_hbm, rew_hbm, rv_hbm, ra_hbm,
                 act_hbm, flat_hbm, nv_hbm,
                 cs_v, sn_v, rew_v, rv_v, ra_v, rows_cs, rows_sn,
                 act_o, flat_o, nv_o, sem1, sem2):
    base = _wid() * _BPW
    sl_in = pl.ds(base, _BPW)
    pltpu.sync_copy(cs_hbm.at[sl_in], cs_v)
    pltpu.sync_copy(sn_hbm.at[sl_in], sn_v)
    c1 = pltpu.async_copy(q_hbm.at[cs_v], rows_cs, sem1)
    c2 = pltpu.async_copy(q_hbm.at[sn_v], rows_sn, sem2)
    pltpu.sync_copy(rew_hbm.at[sl_in], rew_v)
    pltpu.sync_copy(rv_hbm.at[sl_in], rv_v)
    pltpu.sync_copy(ra_hbm.at[sl_in], ra_v)
    c1.wait()
    c2.wait()

    iot = lax.iota(jnp.int32, 16)
    zeros16 = jnp.zeros((16,), jnp.int32)
    def gbody(g, _):
        sl = pl.ds(g * 16, 16)
        rows16 = g * 16 + iot
        m = plsc.load_gather(rows_cs, [rows16, zeros16])
        m2 = plsc.load_gather(rows_sn, [rows16, zeros16])
        mi = zeros16

        for j in range(1, _N_ACTIONS):
            cj = jnp.full((16,), j, jnp.int32)
            v = plsc.load_gather(rows_cs, [rows16, cj])
            v2 = plsc.load_gather(rows_sn, [rows16, cj])
            better = v > m
            m = jnp.where(better, v, m)
            mi = jnp.where(better, cj, mi)
            m2 = jnp.maximum(m2, v2)

        acts = jnp.where(rv_v[sl] > _EPS, mi, ra_v[sl])
        q_sa = plsc.load_gather(rows_cs, [rows16, acts])
        nv = q_sa + _ALPHA * (rew_v[sl] + _GAMMA * m2 - q_sa)
        act_o[sl] = acts
        flat_o[sl] = cs_v[sl] * _N_ACTIONS + acts
        nv_o[sl] = nv
        return 0

    lax.fori_loop(0, _G, gbody, 0)

    pltpu.sync_copy(act_o, act_hbm.at[sl_in])
    pltpu.sync_copy(flat_o, flat_hbm.at[sl_in])
    pltpu.sync_copy(nv_o, nv_hbm.at[sl_in])


def _stage2_body(q_hbm, flat_hbm, nv_hbm, out_hbm,
                 flat_v, val_v, loc_list, val_list, buf0, buf1,
                 semi0, semi1, semo0, semo1):
    wid = _wid()
    row0 = wid * _RPW
    lo = row0 * _N_ACTIONS


    pltpu.sync_copy(flat_hbm, flat_v)
    pltpu.sync_copy(nv_hbm, val_v)

    iot = lax.iota(jnp.int32, 16)
    neg1 = jnp.full((16,), -1, jnp.int32)

    def prebody(i, _):
        loc_list[pl.ds(i * 16, 16)] = neg1
        return 0

    lax.fori_loop(0, _NLV, prebody, 0)

    def fbody(i, cur):
        sl = pl.ds(i * 16, 16)
        fv = flat_v[sl]
        vv = val_v[sl]
        msk = (fv >= lo) & (fv < lo + _RPW * _N_ACTIONS)
        mi32 = msk.astype(jnp.int32)
        cum = plsc.cumsum(mi32)
        pos = jnp.clip(cur + cum - 1, 0, _CAP - 1)
        plsc.store_scatter(loc_list, [pos], fv - lo, mask=msk)
        plsc.store_scatter(val_list, [pos], vv, mask=msk)
        return cur + cum[15]

    cnt = lax.fori_loop(0, _B // 16, fbody, jnp.int32(0))
    nvec = (cnt + 15) >> 4

    def kbody(i, _):
        sl = pl.ds(i * 16, 16)
        lv = loc_list[sl]
        dup = lv < -1
        for s in range(1, 16):
            sh = _dyn_gather(lv, jnp.minimum(iot + s, 15))
            dup = dup | ((lv == sh) & (iot < 16 - s))
        loc_list[sl] = jnp.where(dup, neg1, lv)
        return 0

    lax.fori_loop(0, nvec, kbody, 0)

    bufs = (buf0, buf1)
    semis = (semi0, semi1)
    semos = (semo0, semo1)
    cflat = _CR * _N_ACTIONS

    def rows_at(c):
        return pl.ds(row0 + c * _CR, _CR)

    in_cp = [None, None]
    out_cp = [None, None]
    in_cp[0] = pltpu.async_copy(q_hbm.at[rows_at(0)], bufs[0], semis[0])
    for c in range(_NCH):
        p = c % 2
        if c + 1 < _NCH:
            nxt = (c + 1) % 2
            if out_cp[nxt] is not None:
                out_cp[nxt].wait()
            in_cp[nxt] = pltpu.async_copy(
                q_hbm.at[rows_at(c + 1)], bufs[nxt], semis[nxt])
        in_cp[p].wait()

        c0 = c * cflat
        buf = bufs[p]

        def pbody(i, _, c0=c0, buf=buf):
            sl = pl.ds(i * 16, 16)
            lv = loc_list[sl]
            vv = val_list[sl]
            msk = (lv >= c0) & (lv < c0 + cflat)
            off = jnp.clip(lv - c0, 0, cflat - 1)
            plsc.store_scatter(buf, [off >> 6, off & 63], vv, mask=msk)
            return 0

        lax.fori_loop(0, nvec, pbody, 0)
        out_cp[p] = pltpu.async_copy(buf, out_hbm.at[rows_at(c)], semos[p])
    out_cp[(_NCH - 1) % 2].wait()
    if _NCH > 1:
        out_cp[_NCH % 2].wait()


def kernel(Q_table, reward, rand_vals, current_state, state_next, rand_actions):
    stage1 = functools.partial(
        pl.kernel,
        out_type=(jax.ShapeDtypeStruct((_B,), jnp.int32),
                  jax.ShapeDtypeStruct((_B,), jnp.int32),
                  jax.ShapeDtypeStruct((_B,), jnp.float32)),
        mesh=_mesh(),
        compiler_params=pltpu.CompilerParams(needs_layout_passes=False, use_tc_tiling_on_sc=False),
        scratch_types=[
            pltpu.VMEM((_BPW,), jnp.int32),
            pltpu.VMEM((_BPW,), jnp.int32),
            pltpu.VMEM((_BPW,), jnp.float32),
            pltpu.VMEM((_BPW,), jnp.float32),
            pltpu.VMEM((_BPW,), jnp.int32),
            pltpu.VMEM((_BPW, _N_ACTIONS), jnp.float32),
            pltpu.VMEM((_BPW, _N_ACTIONS), jnp.float32),
            pltpu.VMEM((_BPW,), jnp.int32),
            pltpu.VMEM((_BPW,), jnp.int32),
            pltpu.VMEM((_BPW,), jnp.float32),
            pltpu.SemaphoreType.DMA,
            pltpu.SemaphoreType.DMA,
        ],
    )(_stage1_body)
    actions, flat_idx, new_vals = stage1(
        Q_table, current_state, state_next, reward, rand_vals, rand_actions)

    stage2 = functools.partial(
        pl.kernel,
        out_type=jax.ShapeDtypeStruct((_N_STATES, _N_ACTIONS), jnp.float32),
        mesh=_mesh(),
        compiler_params=pltpu.CompilerParams(needs_layout_passes=False, use_tc_tiling_on_sc=False),
        scratch_types=[
            pltpu.VMEM((_B,), jnp.int32),
            pltpu.VMEM((_B,), jnp.float32),
            pltpu.VMEM((_CAP,), jnp.int32),
            pltpu.VMEM((_CAP,), jnp.float32),
            pltpu.VMEM((_CR, _N_ACTIONS), jnp.float32),
            pltpu.VMEM((_CR, _N_ACTIONS), jnp.float32),
            pltpu.SemaphoreType.DMA,
            pltpu.SemaphoreType.DMA,
            pltpu.SemaphoreType.DMA,
            pltpu.SemaphoreType.DMA,
        ],
    )(_stage2_body)
    new_Q = stage2(Q_table, flat_idx, new_vals)
    return actions, new_Q
